# baseline (speedup 1.0000x reference)
import functools

import jax
import jax.numpy as jnp
from jax import lax
from jax.experimental import pallas as pl
from jax.experimental.pallas import tpu as pltpu
from jax.experimental.pallas import tpu_sc as plsc

_CH = 128


@functools.partial(jax.jit, static_argnames=())
def kernel(input_ids, weight):
    B, T = input_ids.shape
    V, D = weight.shape
    n = B * T
    n_chunks = n // _CH
    nw = 32
    per_w = n_chunks // nw

    ids = input_ids.reshape(n_chunks, _CH).astype(jnp.int32)

    mesh = plsc.VectorSubcoreMesh(core_axis_name="c", subcore_axis_name="s")

    @functools.partial(
        pl.kernel,
        mesh=mesh,
        compiler_params=pltpu.CompilerParams(use_tc_tiling_on_sc=False),
        out_type=jax.ShapeDtypeStruct((n_chunks, _CH, D), jnp.float32),
        scratch_types=[
            pltpu.VMEM((per_w, _CH), jnp.int32),
            pltpu.VMEM((_CH, D), jnp.float32),
            pltpu.SemaphoreType.DMA,
        ],
    )
    def emb(ids_hbm, w_hbm, out_hbm, idx_v, buf, gsem):
        wid = lax.axis_index("s") * 2 + lax.axis_index("c")
        base = wid * per_w
        pltpu.sync_copy(ids_hbm.at[pl.ds(base, per_w)], idx_v)

        def body(j, carry):
            pltpu.async_copy(w_hbm.at[idx_v.at[j]], buf, gsem).wait()
            pltpu.sync_copy(buf, out_hbm.at[base + j])
            return carry

        lax.fori_loop(0, per_w, body, 0)

    out = emb(ids, weight)
    return out.reshape(B, T, D)


# speedup vs baseline: 1.1179x; 1.1179x over previous
import functools

import jax
import jax.numpy as jnp
from jax import lax
from jax.experimental import pallas as pl
from jax.experimental.pallas import tpu as pltpu
from jax.experimental.pallas import tpu_sc as plsc

_CH = 128
_K = 4
_NW = 32


@jax.jit
def kernel(input_ids, weight):
    B, T = input_ids.shape
    V, D = weight.shape
    n = B * T
    n_chunks = n // _CH
    per_w = n_chunks // _NW
    n_grp = per_w // _K
    assert n_chunks % _NW == 0 and per_w % _K == 0 and n_grp % 2 == 0

    ids = input_ids.reshape(n_chunks, _CH).astype(jnp.int32)

    mesh = plsc.VectorSubcoreMesh(core_axis_name="c", subcore_axis_name="s")

    @functools.partial(
        pl.kernel,
        mesh=mesh,
        compiler_params=pltpu.CompilerParams(use_tc_tiling_on_sc=False),
        out_type=jax.ShapeDtypeStruct((_NW * n_grp, _K * _CH, D), jnp.float32),
        scratch_types=[
            pltpu.VMEM((per_w, _CH), jnp.int32),
            pltpu.VMEM((_K * _CH, D), jnp.float32),
            pltpu.VMEM((_K * _CH, D), jnp.float32),
            pltpu.SemaphoreType.DMA,
            pltpu.SemaphoreType.DMA,
            pltpu.SemaphoreType.DMA,
            pltpu.SemaphoreType.DMA,
        ],
    )
    def emb(ids_hbm, w_hbm, out_hbm, idx_v, slab0, slab1, g0, g1, o0, o1):
        wid = lax.axis_index("s") * 2 + lax.axis_index("c")
        cbase = wid * per_w
        obase = wid * n_grp
        slabs = (slab0, slab1)
        gsem = (g0, g1)
        osem = (o0, o1)

        pltpu.sync_copy(ids_hbm.at[pl.ds(cbase, per_w)], idx_v)

        def fire(grp, p):
            for k in range(_K):
                pltpu.async_copy(
                    w_hbm.at[idx_v.at[grp * _K + k]],
                    slabs[p].at[pl.ds(k * _CH, _CH)],
                    gsem[p],
                )

        def drain_and_out(grp, p):
            for k in range(_K):
                pltpu.make_async_copy(
                    w_hbm.at[idx_v.at[grp * _K + k]],
                    slabs[p].at[pl.ds(k * _CH, _CH)],
                    gsem[p],
                ).wait()
            return pltpu.async_copy(slabs[p], out_hbm.at[obase + grp], osem[p])

        fire(0, 0)
        fire(1, 1)

        def body(g2, carry):
            for p in range(2):
                grp = g2 * 2 + p
                out_cp = drain_and_out(grp, p)
                out_cp.wait()
                fire(grp + 2, p)
            return carry

        lax.fori_loop(0, (n_grp - 2) // 2, body, 0)

        drain_and_out(n_grp - 2, 0).wait()
        drain_and_out(n_grp - 1, 1).wait()

    out = emb(ids, weight)
    return out.reshape(B, T, D)
